# baseline (speedup 1.0000x reference)
import functools

import jax
import jax.numpy as jnp
from jax import lax
from jax.experimental import pallas as pl
from jax.experimental.pallas import tpu as pltpu
from jax.experimental.pallas import tpu_sc as plsc

D = 64
SCALE = 8.0

B_TOTAL = 4096 * 200
NW = 32
B_PER_W = B_TOTAL // NW
CHUNK = 512
GSZ = 128
N_GATHER = CHUNK // GSZ
N_CHUNKS = B_PER_W // CHUNK
NBUF = 2


def _make_kernel():
  mesh = plsc.VectorSubcoreMesh(core_axis_name="c", subcore_axis_name="s")

  @functools.partial(
      pl.kernel,
      mesh=mesh,
      compiler_params=pltpu.CompilerParams(use_tc_tiling_on_sc=False),
      out_type=jax.ShapeDtypeStruct((B_TOTAL, 2 * D), jnp.float32),
      scratch_types=[
          pltpu.VMEM((NBUF, CHUNK), jnp.int32),
          pltpu.VMEM((NBUF, CHUNK, D), jnp.float32),
          pltpu.SemaphoreType.DMA,
          pltpu.SemaphoreType.DMA,
          pltpu.SemaphoreType.DMA,
          pltpu.SemaphoreType.DMA,
          pltpu.SemaphoreType.DMA,
          pltpu.SemaphoreType.DMA,
      ],
  )
  def emb_kernel(x_hbm, table_hbm, out_hbm, idx_v, rows_v,
                 si0, si1, sg0, sg1, so0, so1):
    sem_i = (si0, si1)
    sem_g = (sg0, sg1)
    sem_o = (so0, so1)
    wid = lax.axis_index("s") * 2 + lax.axis_index("c")
    wbase = wid * B_PER_W

    def idx_copy(g, b):
      return pltpu.make_async_copy(
          x_hbm.at[pl.ds(wbase + g * CHUNK, CHUNK)], idx_v.at[b], sem_i[b])

    def fire_gather(b):
      for j in range(N_GATHER):
        pltpu.async_copy(
            table_hbm.at[idx_v.at[b, pl.ds(j * GSZ, GSZ)]],
            rows_v.at[b, pl.ds(j * GSZ, GSZ)],
            sem_g[b])

    def drain_gather(b):
      pltpu.make_async_copy(
          out_hbm.at[pl.ds(0, CHUNK), pl.ds(0, D)], rows_v.at[b],
          sem_g[b]).wait()

    def out_copy(g, b):
      return pltpu.make_async_copy(
          rows_v.at[b],
          out_hbm.at[pl.ds(wbase + g * CHUNK, CHUNK), pl.ds(0, D)],
          sem_o[b])

    def scale(b):
      def scale_row(r, c):
        for j in range(D // 16):
          rows_v[b, r, pl.ds(j * 16, 16)] = (
              rows_v[b, r, pl.ds(j * 16, 16)] * SCALE)
        return c

      lax.fori_loop(0, CHUNK, scale_row, 0, unroll=4)

    idx_copy(0, 0).start()
    idx_copy(1, 1).start()
    idx_copy(0, 0).wait()
    fire_gather(0)

    @pl.loop(0, N_CHUNKS, step=NBUF)
    def pipeline(g0):
      for b in range(NBUF):
        g = g0 + b
        nb = (b + 1) % NBUF

        @pl.when(g + 1 < N_CHUNKS)
        def _():
          idx_copy(g + 1, nb).wait()

          @pl.when(g >= 1)
          def _():
            out_copy(g - 1, nb).wait()

          fire_gather(nb)

        drain_gather(b)

        @pl.when(g + 2 < N_CHUNKS)
        def _():
          idx_copy(g + 2, b).start()

        scale(b)
        out_copy(g, b).start()

    out_copy(N_CHUNKS - 1, (N_CHUNKS - 1) % NBUF).wait()

  return emb_kernel


_emb = _make_kernel()

V = 1_000_000
TBW = 4096
N_TBLK = (V + TBW - 1) // TBW


def _transpose_kernel(tt_ref, out_ref):
  t = tt_ref[...].T
  out_ref[...] = jnp.concatenate([t, t], axis=1)


_tpose = pl.pallas_call(
    _transpose_kernel,
    grid=(N_TBLK,),
    in_specs=[pl.BlockSpec((D, TBW), lambda i: (0, i))],
    out_specs=pl.BlockSpec((TBW, 2 * D), lambda i: (i, 0)),
    out_shape=jax.ShapeDtypeStruct((V, 2 * D), jnp.float32),
)


@jax.jit
def kernel(x, table):
  t128 = _tpose(table.T)
  t2 = t128.reshape(2 * V, D)
  xf = x.reshape(-1) * 2
  out = _emb(xf, t2)
  return out[:, :D].reshape(x.shape[0], x.shape[1], D)


# speedup vs baseline: 1.0755x; 1.0755x over previous
import functools

import jax
import jax.numpy as jnp
from jax import lax
from jax.experimental import pallas as pl
from jax.experimental.pallas import tpu as pltpu
from jax.experimental.pallas import tpu_sc as plsc

D = 64
SCALE = 8.0

B_TOTAL = 4096 * 200
NW = 32
B_PER_W = B_TOTAL // NW
CHUNK = 512
GSZ = 128
N_GATHER = CHUNK // GSZ
N_CHUNKS = B_PER_W // CHUNK
NBUF = 2


def _make_kernel():
  mesh = plsc.VectorSubcoreMesh(core_axis_name="c", subcore_axis_name="s")

  @functools.partial(
      pl.kernel,
      mesh=mesh,
      compiler_params=pltpu.CompilerParams(use_tc_tiling_on_sc=False),
      out_type=jax.ShapeDtypeStruct((B_TOTAL, 2 * D), jnp.float32),
      scratch_types=[
          pltpu.VMEM((NBUF, CHUNK), jnp.int32),
          pltpu.VMEM((NBUF, CHUNK, D), jnp.float32),
          pltpu.SemaphoreType.DMA,
          pltpu.SemaphoreType.DMA,
          pltpu.SemaphoreType.DMA,
          pltpu.SemaphoreType.DMA,
          pltpu.SemaphoreType.DMA,
          pltpu.SemaphoreType.DMA,
      ],
  )
  def emb_kernel(x_hbm, table_hbm, out_hbm, idx_v, rows_v,
                 si0, si1, sg0, sg1, so0, so1):
    sem_i = (si0, si1)
    sem_g = (sg0, sg1)
    sem_o = (so0, so1)
    wid = lax.axis_index("s") * 2 + lax.axis_index("c")
    wbase = wid * B_PER_W

    def idx_copy(g, b):
      return pltpu.make_async_copy(
          x_hbm.at[pl.ds(wbase + g * CHUNK, CHUNK)], idx_v.at[b], sem_i[b])

    def fire_gather(b):
      for j in range(N_GATHER):
        pltpu.async_copy(
            table_hbm.at[idx_v.at[b, pl.ds(j * GSZ, GSZ)]],
            rows_v.at[b, pl.ds(j * GSZ, GSZ)],
            sem_g[b])

    def drain_gather(b):
      pltpu.make_async_copy(
          out_hbm.at[pl.ds(0, CHUNK), pl.ds(0, D)], rows_v.at[b],
          sem_g[b]).wait()

    def out_copy(g, b):
      return pltpu.make_async_copy(
          rows_v.at[b],
          out_hbm.at[pl.ds(wbase + g * CHUNK, CHUNK), pl.ds(0, D)],
          sem_o[b])

    def scale(b):
      def scale_row(r, c):
        for j in range(D // 16):
          rows_v[b, r, pl.ds(j * 16, 16)] = (
              rows_v[b, r, pl.ds(j * 16, 16)] * SCALE)
        return c

      lax.fori_loop(0, CHUNK, scale_row, 0, unroll=4)

    idx_copy(0, 0).start()
    idx_copy(1, 1).start()
    idx_copy(0, 0).wait()
    fire_gather(0)

    @pl.loop(0, N_CHUNKS, step=NBUF)
    def pipeline(g0):
      for b in range(NBUF):
        g = g0 + b
        nb = (b + 1) % NBUF

        @pl.when(g + 1 < N_CHUNKS)
        def _():
          idx_copy(g + 1, nb).wait()

          @pl.when(g >= 1)
          def _():
            out_copy(g - 1, nb).wait()

          fire_gather(nb)

        drain_gather(b)

        @pl.when(g + 2 < N_CHUNKS)
        def _():
          idx_copy(g + 2, b).start()

        scale(b)
        out_copy(g, b).start()

    out_copy(N_CHUNKS - 1, (N_CHUNKS - 1) % NBUF).wait()

  return emb_kernel


_emb = _make_kernel()

V = 1_000_000
TBW = 4096
N_TBLK = (V + TBW - 1) // TBW


def _transpose_kernel(tt_ref, out_ref):
  out_ref[:, :D] = tt_ref[...].T


_tpose = pl.pallas_call(
    _transpose_kernel,
    grid=(N_TBLK,),
    in_specs=[pl.BlockSpec((D, TBW), lambda i: (0, i))],
    out_specs=pl.BlockSpec((TBW, 2 * D), lambda i: (i, 0)),
    out_shape=jax.ShapeDtypeStruct((V, 2 * D), jnp.float32),
)


@jax.jit
def kernel(x, table):
  t128 = _tpose(table.T)
  t2 = t128.reshape(2 * V, D)
  xf = x.reshape(-1) * 2
  out = _emb(xf, t2)
  return out[:, :D].reshape(x.shape[0], x.shape[1], D)


# speedup vs baseline: 1.2232x; 1.1373x over previous
import functools

import jax
import jax.numpy as jnp
from jax import lax
from jax.experimental import pallas as pl
from jax.experimental.pallas import tpu as pltpu
from jax.experimental.pallas import tpu_sc as plsc

D = 64
SCALE = 8.0

B_TOTAL = 4096 * 200
NW = 32
B_PER_W = B_TOTAL // NW
CHUNK = 512
GSZ = 128
N_GATHER = CHUNK // GSZ
N_CHUNKS = B_PER_W // CHUNK
NBUF = 2

TBW = 4096
NBLK_D = 124
OFF_B = 121
DENSE_ROWS = NBLK_D * TBW
SPLIT = DENSE_ROWS
SUB = 2 * OFF_B * TBW - 1


def _make_kernel():
  mesh = plsc.VectorSubcoreMesh(core_axis_name="c", subcore_axis_name="s")

  @functools.partial(
      pl.kernel,
      mesh=mesh,
      compiler_params=pltpu.CompilerParams(use_tc_tiling_on_sc=False),
      out_type=jax.ShapeDtypeStruct((B_TOTAL, 2 * D), jnp.float32),
      scratch_types=[
          pltpu.VMEM((NBUF, CHUNK), jnp.int32),
          pltpu.VMEM((NBUF, CHUNK, D), jnp.float32),
          pltpu.SemaphoreType.DMA,
          pltpu.SemaphoreType.DMA,
          pltpu.SemaphoreType.DMA,
          pltpu.SemaphoreType.DMA,
          pltpu.SemaphoreType.DMA,
          pltpu.SemaphoreType.DMA,
      ],
  )
  def emb_kernel(x_hbm, table_hbm, out_hbm, idx_v, rows_v,
                 si0, si1, sg0, sg1, so0, so1):
    sem_i = (si0, si1)
    sem_g = (sg0, sg1)
    sem_o = (so0, so1)
    wid = lax.axis_index("s") * 2 + lax.axis_index("c")
    wbase = wid * B_PER_W

    def idx_copy(g, b):
      return pltpu.make_async_copy(
          x_hbm.at[pl.ds(wbase + g * CHUNK, CHUNK)], idx_v.at[b], sem_i[b])

    def fire_gather(b):
      for j in range(N_GATHER):
        pltpu.async_copy(
            table_hbm.at[idx_v.at[b, pl.ds(j * GSZ, GSZ)]],
            rows_v.at[b, pl.ds(j * GSZ, GSZ)],
            sem_g[b])

    def drain_gather(b):
      pltpu.make_async_copy(
          out_hbm.at[pl.ds(0, CHUNK), pl.ds(0, D)], rows_v.at[b],
          sem_g[b]).wait()

    def out_copy(g, b):
      return pltpu.make_async_copy(
          rows_v.at[b],
          out_hbm.at[pl.ds(wbase + g * CHUNK, CHUNK), pl.ds(0, D)],
          sem_o[b])

    def transform(b):
      def tr(k, c):
        iv = idx_v[b, pl.ds(k * 16, 16)]
        idx_v[b, pl.ds(k * 16, 16)] = (
            iv * 2 - jnp.where(iv >= SPLIT, SUB, 0))
        return c

      lax.fori_loop(0, CHUNK // 16, tr, 0, unroll=4)

    def scale(b):
      def scale_row(r, c):
        for j in range(D // 16):
          rows_v[b, r, pl.ds(j * 16, 16)] = (
              rows_v[b, r, pl.ds(j * 16, 16)] * SCALE)
        return c

      lax.fori_loop(0, CHUNK, scale_row, 0, unroll=4)

    idx_copy(0, 0).start()
    idx_copy(1, 1).start()
    idx_copy(0, 0).wait()
    transform(0)
    fire_gather(0)

    @pl.loop(0, N_CHUNKS, step=NBUF)
    def pipeline(g0):
      for b in range(NBUF):
        g = g0 + b
        nb = (b + 1) % NBUF

        @pl.when(g + 1 < N_CHUNKS)
        def _():
          idx_copy(g + 1, nb).wait()

          @pl.when(g >= 1)
          def _():
            out_copy(g - 1, nb).wait()

          transform(nb)
          fire_gather(nb)

        drain_gather(b)

        @pl.when(g + 2 < N_CHUNKS)
        def _():
          idx_copy(g + 2, b).start()

        scale(b)
        out_copy(g, b).start()

    out_copy(N_CHUNKS - 1, (N_CHUNKS - 1) % NBUF).wait()

  return emb_kernel


_emb = _make_kernel()

V = 1_000_000


def _transpose_kernel(lo_ref, hi_ref, out_ref):
  out_ref[:, :D] = lo_ref[...].T
  out_ref[:, D:] = hi_ref[...].T


_tpose = pl.pallas_call(
    _transpose_kernel,
    grid=(NBLK_D,),
    in_specs=[
        pl.BlockSpec((D, TBW), lambda i: (0, i)),
        pl.BlockSpec((D, TBW), lambda i: (0, i + OFF_B)),
    ],
    out_specs=pl.BlockSpec((TBW, 2 * D), lambda i: (i, 0)),
    out_shape=jax.ShapeDtypeStruct((DENSE_ROWS, 2 * D), jnp.float32),
)


@jax.jit
def kernel(x, table):
  tt = table.T
  t128 = _tpose(tt, tt)
  t2 = t128.reshape(2 * DENSE_ROWS, D)
  xf = x.reshape(-1)
  out = _emb(xf, t2)
  return out[:, :D].reshape(x.shape[0], x.shape[1], D)
